# Initial kernel scaffold; baseline (speedup 1.0000x reference)
import jax
import jax.numpy as jnp
from jax.experimental import pallas as pl


def kernel(x, idx, emb_table):
    raise NotImplementedError("write your pallas kernel here")


# speedup vs baseline: 215.1661x; 215.1661x over previous
import jax
import jax.numpy as jnp
from jax.experimental import pallas as pl
from jax.experimental.pallas import tpu as pltpu

MINPOS = -32
NBIN = 65
R = 8


def _body(x_ref, idxL_ref, idxc_ref, tab_ref, o_ref):
    jcol = idxL_ref[0]
    L = jcol.shape[0]
    tab = tab_ref[...]
    lanes = jax.lax.broadcasted_iota(jnp.int32, (L, NBIN), 1)
    for i in range(R):
        vi = idxc_ref[0, i, 0]
        ib = jnp.clip(jcol - vi - MINPOS, 0, NBIN - 1)
        oh = (ib == lanes).astype(jnp.float32)
        emb = jnp.dot(oh, tab, preferred_element_type=jnp.float32)
        o_ref[0, i] = x_ref[0, i] + emb


def kernel(x, idx, emb_table):
    B, L, _, D = x.shape
    idx32 = idx.astype(jnp.int32)
    idx_colL = idx32.reshape(B, L, 1)
    grid = (B, L // R)
    return pl.pallas_call(
        _body,
        grid=grid,
        in_specs=[
            pl.BlockSpec((1, R, L, D), lambda b, r: (b, r, 0, 0)),
            pl.BlockSpec((1, L, 1), lambda b, r: (b, 0, 0)),
            pl.BlockSpec((1, R, 1), lambda b, r: (b, r, 0)),
            pl.BlockSpec((NBIN, D), lambda b, r: (0, 0)),
        ],
        out_specs=pl.BlockSpec((1, R, L, D), lambda b, r: (b, r, 0, 0)),
        out_shape=jax.ShapeDtypeStruct(x.shape, x.dtype),
        compiler_params=pltpu.CompilerParams(
            dimension_semantics=("parallel", "arbitrary"),
        ),
    )(x, idx_colL, idx_colL, emb_table)
